# Initial kernel scaffold; baseline (speedup 1.0000x reference)
import functools

import jax
import jax.numpy as jnp
from jax import lax
from jax.experimental import pallas as pl
from jax.experimental.pallas import tpu as pltpu
from jax.experimental.pallas import tpu_sc as plsc

HIDDEN = 64
SEQ = 200
BATCH = 4096
EPS = 1e-12

NC = 2
NS = 16
NW = NC * NS
ROWS_PER_W = BATCH // NW
L = 16
NV = HIDDEN // L


def _rsqrt(v):
    i = plsc.bitcast(v, jnp.int32)
    y = plsc.bitcast(jnp.int32(0x5F3759DF) - (i >> 1), jnp.float32)
    for _ in range(2):
        y = y * (1.5 - 0.5 * v * y * y)
    return y


def _body(seq_hbm, word_hbm, pos_hbm, gamma_hbm, beta_hbm, out_hbm,
          idx_v, rows_v, pos_v, gam_v, bet_v, sem):
    wid = lax.axis_index("s") * NC + lax.axis_index("c")
    row0 = wid * ROWS_PER_W

    pltpu.sync_copy(pos_hbm, pos_v)
    pltpu.sync_copy(gamma_hbm, gam_v)
    pltpu.sync_copy(beta_hbm, bet_v)
    gam = [gam_v[pl.ds(L * j, L)] for j in range(NV)]
    bet = [bet_v[pl.ds(L * j, L)] for j in range(NV)]

    def do_row(r, _):
        row = row0 + r
        pltpu.sync_copy(seq_hbm.at[row], idx_v)
        d0 = pltpu.async_copy(word_hbm.at[idx_v.at[0]],
                              rows_v.at[pl.ds(0, 100)], sem)
        d1 = pltpu.async_copy(word_hbm.at[idx_v.at[1]],
                              rows_v.at[pl.ds(100, 100)], sem)
        d0.wait()
        d1.wait()

        def do_tok(t, _):
            x = [rows_v[t, pl.ds(L * j, L)] + pos_v[t, pl.ds(L * j, L)]
                 for j in range(NV)]
            s = (x[0] + x[1]) + (x[2] + x[3])
            q = (x[0] * x[0] + x[1] * x[1]) + (x[2] * x[2] + x[3] * x[3])
            mean = jnp.full((L,), jnp.sum(s), jnp.float32) * (1.0 / HIDDEN)
            ex2 = jnp.full((L,), jnp.sum(q), jnp.float32) * (1.0 / HIDDEN)
            var = ex2 - mean * mean
            rstd = _rsqrt(var + EPS)
            for j in range(NV):
                m = gam[j] * rstd
                rows_v[t, pl.ds(L * j, L)] = x[j] * m + (bet[j] - mean * m)
            return 0

        lax.fori_loop(0, SEQ, do_tok, 0)
        pltpu.sync_copy(rows_v, out_hbm.at[pl.ds(row * SEQ, SEQ)])
        return 0

    lax.fori_loop(0, ROWS_PER_W, do_row, 0)


def kernel(seq, word_table, pos_table, gamma, beta):
    seq3 = seq.reshape(BATCH, 2, SEQ // 2).astype(jnp.int32)
    mesh = plsc.VectorSubcoreMesh(core_axis_name="c", subcore_axis_name="s",
                                  num_cores=NC, num_subcores=NS)
    k = pl.kernel(
        _body,
        out_type=jax.ShapeDtypeStruct((BATCH * SEQ, HIDDEN), jnp.float32),
        mesh=mesh,
        scratch_types=[
            pltpu.VMEM((2, SEQ // 2), jnp.int32),
            pltpu.VMEM((SEQ, HIDDEN), jnp.float32),
            pltpu.VMEM((SEQ, HIDDEN), jnp.float32),
            pltpu.VMEM((HIDDEN,), jnp.float32),
            pltpu.VMEM((HIDDEN,), jnp.float32),
            pltpu.SemaphoreType.DMA,
        ],
    )
    out = k(seq3, word_table, pos_table, gamma, beta)
    return out.reshape(BATCH, SEQ, HIDDEN)


# speedup vs baseline: 2.7586x; 2.7586x over previous
import jax
import jax.numpy as jnp
from jax import lax
from jax.experimental import pallas as pl
from jax.experimental.pallas import tpu as pltpu
from jax.experimental.pallas import tpu_sc as plsc

HIDDEN = 64
SEQ = 200
BATCH = 4096
EPS = 1e-12

NC = 2
NS = 16
NW = NC * NS
ROWS_PER_W = BATCH // NW
L = 16
NV = HIDDEN // L

_GATHER_DNUMS = lax.GatherDimensionNumbers(
    offset_dims=(), collapsed_slice_dims=(0,), start_index_map=(0,))


def _lane_perm(x, perm):
    return lax.gather(x, perm[:, None], dimension_numbers=_GATHER_DNUMS,
                      slice_sizes=(1,),
                      mode=lax.GatherScatterMode.PROMISE_IN_BOUNDS)


def _allsum(x):
    lane = lax.iota(jnp.int32, L)
    for sh in (8, 4, 2, 1):
        x = x + _lane_perm(x, lane ^ sh)
    return x


def _rsqrt(v):
    i = lax.bitcast_convert_type(v, jnp.int32)
    y = lax.bitcast_convert_type(jnp.int32(0x5F3759DF) - (i >> 1),
                                 jnp.float32)
    for _ in range(2):
        y = y * (1.5 - 0.5 * v * y * y)
    return y


def _body(seq_hbm, word_hbm, pos_hbm, gamma_hbm, beta_hbm, out_hbm,
          idx_v, rows_v, pos_v, gam_v, bet_v, sg0, sg1, so0, so1):
    wid = lax.axis_index("s") * NC + lax.axis_index("c")
    row0 = wid * ROWS_PER_W
    sems_g = (sg0, sg1)
    sems_o = (so0, so1)

    pltpu.sync_copy(pos_hbm, pos_v)
    pltpu.sync_copy(gamma_hbm, gam_v)
    pltpu.sync_copy(beta_hbm, bet_v)
    pltpu.sync_copy(seq_hbm.at[wid], idx_v)
    gam = [gam_v[pl.ds(L * j, L)] for j in range(NV)]
    bet = [bet_v[pl.ds(L * j, L)] for j in range(NV)]

    def fire_gather(r, b):
        pltpu.async_copy(word_hbm.at[idx_v.at[2 * r]],
                         rows_v.at[b, pl.ds(0, 100)], sems_g[b])
        pltpu.async_copy(word_hbm.at[idx_v.at[2 * r + 1]],
                         rows_v.at[b, pl.ds(100, 100)], sems_g[b])

    def wait_gather(b):
        pltpu.make_async_copy(out_hbm.at[pl.ds(0, SEQ)], rows_v.at[b],
                              sems_g[b]).wait()

    def fire_out(r, b):
        pltpu.async_copy(rows_v.at[b], out_hbm.at[pl.ds((row0 + r) * SEQ,
                                                        SEQ)], sems_o[b])

    def wait_out(b):
        pltpu.make_async_copy(rows_v.at[b], out_hbm.at[pl.ds(0, SEQ)],
                              sems_o[b]).wait()

    def compute(b):
        @plsc.parallel_loop(0, SEQ, 1, unroll=4)
        def _(t):
            x = [rows_v[b, t, pl.ds(L * j, L)] + pos_v[t, pl.ds(L * j, L)]
                 for j in range(NV)]
            s = (x[0] + x[1]) + (x[2] + x[3])
            q = (x[0] * x[0] + x[1] * x[1]) + (x[2] * x[2] + x[3] * x[3])
            mean = _allsum(s) * (1.0 / HIDDEN)
            ex2 = _allsum(q) * (1.0 / HIDDEN)
            var = ex2 - mean * mean
            rstd = _rsqrt(var + EPS)
            for j in range(NV):
                m = gam[j] * rstd
                rows_v[b, t, pl.ds(L * j, L)] = x[j] * m + (bet[j] - mean * m)

    fire_gather(0, 0)

    def iter_k(k, _):
        r = 2 * k

        @pl.when(k > 0)
        def _():
            wait_out(1)
        fire_gather(r + 1, 1)
        wait_gather(0)
        compute(0)
        fire_out(r, 0)

        @pl.when(k < ROWS_PER_W // 2 - 1)
        def _():
            wait_out(0)
            fire_gather(r + 2, 0)
        wait_gather(1)
        compute(1)
        fire_out(r + 1, 1)
        return 0

    lax.fori_loop(0, ROWS_PER_W // 2, iter_k, 0)
    wait_out(0)
    wait_out(1)


def kernel(seq, word_table, pos_table, gamma, beta):
    seq3 = seq.reshape(NW, 2 * ROWS_PER_W, SEQ // 2).astype(jnp.int32)
    mesh = plsc.VectorSubcoreMesh(core_axis_name="c", subcore_axis_name="s",
                                  num_cores=NC, num_subcores=NS)
    k = pl.kernel(
        _body,
        out_type=jax.ShapeDtypeStruct((BATCH * SEQ, HIDDEN), jnp.float32),
        mesh=mesh,
        scratch_types=[
            pltpu.VMEM((2 * ROWS_PER_W, SEQ // 2), jnp.int32),
            pltpu.VMEM((2, SEQ, HIDDEN), jnp.float32),
            pltpu.VMEM((SEQ, HIDDEN), jnp.float32),
            pltpu.VMEM((HIDDEN,), jnp.float32),
            pltpu.VMEM((HIDDEN,), jnp.float32),
            pltpu.SemaphoreType.DMA,
            pltpu.SemaphoreType.DMA,
            pltpu.SemaphoreType.DMA,
            pltpu.SemaphoreType.DMA,
        ],
        compiler_params=pltpu.CompilerParams(use_tc_tiling_on_sc=False),
    )
    out = k(seq3, word_table, pos_table, gamma, beta)
    return out.reshape(BATCH, SEQ, HIDDEN)


# speedup vs baseline: 3.2875x; 1.1917x over previous
import jax
import jax.numpy as jnp
from jax import lax
from jax.experimental import pallas as pl
from jax.experimental.pallas import tpu as pltpu
from jax.experimental.pallas import tpu_sc as plsc

HIDDEN = 64
SEQ = 200
BATCH = 4096
EPS = 1e-12

NC = 2
NS = 16
NW = NC * NS
ROWS_PER_W = BATCH // NW
L = 16
NV = HIDDEN // L
HALF = SEQ // 2
PADW = 128

_GATHER_DNUMS = lax.GatherDimensionNumbers(
    offset_dims=(), collapsed_slice_dims=(0,), start_index_map=(0,))


def _lane_perm(x, perm):
    return lax.gather(x, perm[:, None], dimension_numbers=_GATHER_DNUMS,
                      slice_sizes=(1,),
                      mode=lax.GatherScatterMode.PROMISE_IN_BOUNDS)


def _allsum(x):
    lane = lax.iota(jnp.int32, L)
    for sh in (8, 4, 2, 1):
        x = x + _lane_perm(x, lane ^ sh)
    return x


def _rsqrt(v):
    i = lax.bitcast_convert_type(v, jnp.int32)
    y = lax.bitcast_convert_type(jnp.int32(0x5F3759DF) - (i >> 1),
                                 jnp.float32)
    for _ in range(2):
        y = y * (1.5 - 0.5 * v * y * y)
    return y


def _body(seq_hbm, word_hbm, pos_hbm, gamma_hbm, beta_hbm, out_hbm,
          idx_v, rows_v, outs_v, pos_v, gam_v, bet_v, sg0, sg1, so0, so1):
    wid = lax.axis_index("s") * NC + lax.axis_index("c")
    row0 = wid * ROWS_PER_W
    sems_g = (sg0, sg1)
    sems_o = (so0, so1)

    pltpu.sync_copy(pos_hbm, pos_v)
    pltpu.sync_copy(gamma_hbm, gam_v)
    pltpu.sync_copy(beta_hbm, bet_v)
    gam = [gam_v[pl.ds(L * j, L)] for j in range(NV)]
    bet = [bet_v[pl.ds(L * j, L)] for j in range(NV)]

    def stage_idx(r):
        pltpu.sync_copy(seq_hbm.at[2 * (row0 + r)], idx_v.at[0])
        pltpu.sync_copy(seq_hbm.at[2 * (row0 + r) + 1], idx_v.at[1])

    def fire_gather(h):
        pltpu.async_copy(word_hbm.at[idx_v.at[h]], rows_v.at[h], sems_g[h])

    def wait_gather(h):
        pltpu.make_async_copy(word_hbm.at[idx_v.at[h]], rows_v.at[h],
                              sems_g[h]).wait()

    def fire_out(r, b):
        pltpu.async_copy(outs_v.at[b], out_hbm.at[row0 + r], sems_o[b])

    def wait_out(b):
        pltpu.make_async_copy(outs_v.at[b], out_hbm.at[0], sems_o[b]).wait()

    def compute(h, b):
        @plsc.parallel_loop(0, HALF, 1, unroll=8)
        def _(t):
            tt = h * HALF + t
            x = [rows_v[h, t, pl.ds(L * j, L)] + pos_v[tt, pl.ds(L * j, L)]
                 for j in range(NV)]
            s = (x[0] + x[1]) + (x[2] + x[3])
            q = (x[0] * x[0] + x[1] * x[1]) + (x[2] * x[2] + x[3] * x[3])
            mean = _allsum(s) * (1.0 / HIDDEN)
            var = _allsum(q) * (1.0 / HIDDEN) - mean * mean
            rstd = _rsqrt(var + EPS)
            for j in range(NV):
                xn = (x[j] - mean) * rstd
                outs_v[b, tt, pl.ds(L * j, L)] = xn * gam[j] + bet[j]

    stage_idx(0)
    fire_gather(0)
    fire_gather(1)

    def row_step(r, ob):
        @pl.when(r >= 2)
        def _():
            wait_out(ob)
        wait_gather(0)
        compute(0, ob)
        wait_gather(1)

        @pl.when(r < ROWS_PER_W - 1)
        def _():
            stage_idx(r + 1)
            fire_gather(0)
        compute(1, ob)

        @pl.when(r < ROWS_PER_W - 1)
        def _():
            fire_gather(1)
        fire_out(r, ob)

    def iter_k(k, _):
        row_step(2 * k, 0)
        row_step(2 * k + 1, 1)
        return 0

    lax.fori_loop(0, ROWS_PER_W // 2, iter_k, 0)
    wait_out(0)
    wait_out(1)


def kernel(seq, word_table, pos_table, gamma, beta):
    seq2 = seq.reshape(2 * BATCH, SEQ // 2).astype(jnp.int32)
    word128 = jnp.pad(word_table, ((0, 0), (0, PADW - HIDDEN)))
    mesh = plsc.VectorSubcoreMesh(core_axis_name="c", subcore_axis_name="s",
                                  num_cores=NC, num_subcores=NS)
    k = pl.kernel(
        _body,
        out_type=jax.ShapeDtypeStruct((BATCH, SEQ, HIDDEN), jnp.float32),
        mesh=mesh,
        scratch_types=[
            pltpu.VMEM((2, HALF), jnp.int32),
            pltpu.VMEM((2, HALF, PADW), jnp.float32),
            pltpu.VMEM((2, SEQ, HIDDEN), jnp.float32),
            pltpu.VMEM((SEQ, HIDDEN), jnp.float32),
            pltpu.VMEM((HIDDEN,), jnp.float32),
            pltpu.VMEM((HIDDEN,), jnp.float32),
            pltpu.SemaphoreType.DMA,
            pltpu.SemaphoreType.DMA,
            pltpu.SemaphoreType.DMA,
            pltpu.SemaphoreType.DMA,
        ],
        compiler_params=pltpu.CompilerParams(use_tc_tiling_on_sc=True),
    )
    return k(seq2, word128, pos_table, gamma, beta)
